# baseline (speedup 1.0000x reference)
import functools

import jax
import jax.numpy as jnp
from jax import lax
from jax.experimental import pallas as pl
from jax.experimental.pallas import tpu as pltpu
from jax.experimental.pallas import tpu_sc as plsc

_VOCAB = 256
_EMBED = 32
_L = 819200

_NC = 2
_NS = 16
_NW = _NC * _NS
_CHUNK = _L // _NW

_LANES = 16


_STRIDE = _VOCAB + 1
_HALF = _CHUNK // 2


def _sc_hist_body(ids_hbm, out_hbm, ids_v, hist_v, local_v, sem0, sem1):
    wid = lax.axis_index("s") * _NC + lax.axis_index("c")
    base = wid * _CHUNK

    cp0 = pltpu.make_async_copy(
        ids_hbm.at[pl.ds(base, _HALF)], ids_v.at[pl.ds(0, _HALF)], sem0)
    cp1 = pltpu.make_async_copy(
        ids_hbm.at[pl.ds(base + _HALF, _HALF)], ids_v.at[pl.ds(_HALF, _HALF)],
        sem1)
    cp0.start()
    cp1.start()

    zeros16 = jnp.zeros((_LANES,), jnp.float32)
    for j in range(_LANES * _STRIDE // _LANES):
        hist_v[pl.ds(j * _LANES, _LANES)] = zeros16

    lane_off = lax.iota(jnp.int32, _LANES) * _STRIDE
    ones16 = jnp.ones((_LANES,), jnp.float32)

    cp0.wait()

    @plsc.parallel_loop(0, _HALF // _LANES, unroll=16)
    def _scatter0(i):
        ids16 = ids_v[pl.ds(i * _LANES, _LANES)]
        plsc.addupdate_scatter(hist_v, [ids16 + lane_off], ones16)

    cp1.wait()

    @plsc.parallel_loop(_HALF // _LANES, _CHUNK // _LANES, unroll=16)
    def _scatter1(i):
        ids16 = ids_v[pl.ds(i * _LANES, _LANES)]
        plsc.addupdate_scatter(hist_v, [ids16 + lane_off], ones16)

    for c in range(_VOCAB // _LANES):
        acc = hist_v[pl.ds(c * _LANES, _LANES)]
        for r in range(1, _LANES):
            acc = acc + hist_v[pl.ds(r * _STRIDE + c * _LANES, _LANES)]
        local_v[pl.ds(c * _LANES, _LANES)] = acc

    pltpu.sync_copy(local_v, out_hbm.at[wid])


_sc_hist = functools.partial(
    pl.kernel,
    out_type=jax.ShapeDtypeStruct((_NW, _VOCAB), jnp.float32),
    mesh=plsc.VectorSubcoreMesh(core_axis_name="c", subcore_axis_name="s"),
    scratch_types=[
        pltpu.VMEM((_CHUNK,), jnp.int32),
        pltpu.VMEM((_LANES * _STRIDE,), jnp.float32),
        pltpu.VMEM((_VOCAB,), jnp.float32),
        pltpu.SemaphoreType.DMA,
        pltpu.SemaphoreType.DMA,
    ],
    compiler_params=pltpu.CompilerParams(needs_layout_passes=False),
)(_sc_hist_body)


def _tc_finish_body(partials_ref, table_ref, out_ref):
    counts = jnp.sum(partials_ref[...], axis=0)
    out_ref[...] = jnp.sum(
        counts[:, None] * table_ref[...], axis=0, keepdims=True
    ) * (1.0 / _L)


def kernel(goal, table):
    partials = _sc_hist(goal)
    out = pl.pallas_call(
        _tc_finish_body,
        out_shape=jax.ShapeDtypeStruct((1, _EMBED), jnp.float32),
    )(partials, table)
    return out.reshape(_EMBED)
